# baseline (speedup 1.0000x reference)
import functools

import jax
import jax.numpy as jnp
from jax import lax
from jax.experimental import pallas as pl
from jax.experimental.pallas import tpu as pltpu
from jax.experimental.pallas import tpu_sc as plsc

DIM = 64
B = 4096
L = 200
NW = 32
TOTAL = B * L
PER_W = TOTAL // NW
CHUNK = 1024
IDXW = 128
NIDX = CHUNK // IDXW
NCHUNKS = PER_W // CHUNK


def _body(x_hbm, lens_hbm, table_hbm, out_hbm, idx_v, rows_v, mask_v, lens_v, gsem):
    c = lax.axis_index("c")
    s = lax.axis_index("s")
    wid = s * 2 + c
    base = wid * PER_W

    pltpu.sync_copy(lens_hbm, lens_v)

    lane = lax.iota(jnp.int32, 16)

    b0 = wid * (PER_W // L)

    def chunk_body(g, carry):
        b_c, l_c = carry
        off = base + g * CHUNK
        pltpu.sync_copy(x_hbm.at[pl.ds(pl.multiple_of(off // IDXW, 8), NIDX)], idx_v)
        copies = [
            pltpu.async_copy(
                table_hbm.at[idx_v.at[j]],
                rows_v.at[pl.ds(j * IDXW, IDXW)],
                gsem,
            )
            for j in range(NIDX)
        ]

        def mask_body(i, bl):
            b_s, l_s = bl
            lvec = l_s + lane
            wrap = jnp.where(lvec >= L, 1, 0)
            bvec = b_s + wrap
            pos = lvec - wrap * L
            lv = plsc.load_gather(lens_v, [bvec])
            mask_v[pl.ds(i * 16, 16)] = jnp.where(pos < lv, 1.0, 0.0).astype(jnp.float32)
            l_n = l_s + 16
            w = jnp.where(l_n >= L, 1, 0)
            return (b_s + w, l_n - w * L)

        b_c, l_c = lax.fori_loop(0, CHUNK // 16, mask_body, (b_c, l_c))
        for cp in copies:
            cp.wait()

        def row_body(r, _):
            m = plsc.load_gather(mask_v, [jnp.full((16,), r, dtype=jnp.int32)])
            for j in range(DIM // 16):
                d = rows_v[r, pl.ds(j * 16, 16)]
                rows_v[r, pl.ds(j * 16, 16)] = jnp.maximum(d, 0.0) * m
            return 0

        lax.fori_loop(0, CHUNK, row_body, 0, unroll=2)

        pltpu.sync_copy(rows_v, out_hbm.at[pl.ds(off, CHUNK)])
        return (b_c, l_c)

    lax.fori_loop(0, NCHUNKS, chunk_body, (b0, jnp.int32(0)))


@jax.jit
def _run(x2, x_lens, table):
    mesh = plsc.VectorSubcoreMesh(core_axis_name="c", subcore_axis_name="s")
    k = functools.partial(
        pl.kernel,
        mesh=mesh,
        out_type=jax.ShapeDtypeStruct((TOTAL, DIM), jnp.float32),
        scratch_types=[
            pltpu.VMEM((NIDX, IDXW), jnp.int32),
            pltpu.VMEM((CHUNK, DIM), jnp.float32),
            pltpu.VMEM((CHUNK,), jnp.float32),
            pltpu.VMEM((B,), jnp.int32),
            pltpu.SemaphoreType.DMA,
        ],
        compiler_params=pltpu.CompilerParams(use_tc_tiling_on_sc=False, needs_layout_passes=False),
    )(_body)
    return k(x2, x_lens, table)


def kernel(x, x_lens, table):
    x2 = x.reshape(TOTAL // IDXW, IDXW)
    out = _run(x2, x_lens, table)
    return out.reshape(B, L, DIM)
